# baseline (speedup 1.0000x reference)
import functools

import jax
import jax.numpy as jnp
from jax import lax
from jax.experimental import pallas as pl
from jax.experimental.pallas import tpu as pltpu
from jax.experimental.pallas import tpu_sc as plsc

C = 1024
E = 8
HID = 4096
N = 4096
CAP = 640
PAD = 768
TT = 512
NT = N // TT
TH = 2048
NH = HID // TH
SUBS = ((0, 256), (256, 256), (512, 128))

CHUNK = 32
NW = 32
PER_W = N // NW
GW = 128


def _router_body(x_ref, wr_ref, br_ref,
                 dst_ref, gdst_ref, val_ref, cnt_ref, aux_ref,
                 cnt_acc, imp_acc, tril_s):
    t = pl.program_id(0)

    @pl.when(t == 0)
    def _():
        cnt_acc[...] = jnp.zeros_like(cnt_acc)
        imp_acc[...] = jnp.zeros_like(imp_acc)
        ri = lax.broadcasted_iota(jnp.int32, (TT, TT), 0)
        ci = lax.broadcasted_iota(jnp.int32, (TT, TT), 1)
        tril_s[...] = (ri > ci).astype(jnp.float32)

    logits = jnp.dot(x_ref[...], wr_ref[...],
                     preferred_element_type=jnp.float32) + br_ref[...]
    m = jnp.max(logits, axis=1, keepdims=True)
    ex = jnp.exp(logits - m)
    probs = ex / jnp.sum(ex, axis=1, keepdims=True)
    pmax = jnp.max(probs, axis=1, keepdims=True)
    eids = lax.broadcasted_iota(jnp.int32, (TT, E), 1)
    idx = jnp.min(jnp.where(probs == pmax, eids, E), axis=1)
    mask = (eids == idx[:, None]).astype(jnp.float32)

    prev = jnp.dot(tril_s[...], mask,
                   preferred_element_type=jnp.float32) + cnt_acc[...]
    pos = jnp.sum(prev * mask, axis=1).astype(jnp.int32)

    cnt_acc[...] = cnt_acc[...] + jnp.sum(mask, axis=0, keepdims=True)
    imp_acc[...] = imp_acc[...] + jnp.sum(probs, axis=0, keepdims=True)

    kept = pos < CAP
    dst_ref[...] = (idx * PAD + jnp.where(kept, pos, CAP)).reshape(1, 1, TT)
    gdst_ref[...] = (idx * PAD + jnp.where(kept, pos, CAP + 1)).reshape(1, 1, TT)
    val_ref[...] = jnp.broadcast_to(pmax, (TT, GW))

    @pl.when(t == NT - 1)
    def _():
        cnt_k = jnp.minimum(cnt_acc[...], float(CAP))
        cnt_ref[...] = cnt_k.astype(jnp.int32)
        aux = jnp.sum((imp_acc[...] / N) * (cnt_k / N)) * E
        aux_ref[...] = aux.reshape(1, 1)


def _router(x_flat, Wr, br2):
    return pl.pallas_call(
        _router_body,
        grid=(NT,),
        in_specs=[
            pl.BlockSpec((TT, C), lambda t: (t, 0)),
            pl.BlockSpec((C, E), lambda t: (0, 0)),
            pl.BlockSpec((1, E), lambda t: (0, 0)),
        ],
        out_specs=[
            pl.BlockSpec((1, 1, TT), lambda t: (t, 0, 0)),
            pl.BlockSpec((1, 1, TT), lambda t: (t, 0, 0)),
            pl.BlockSpec((TT, GW), lambda t: (t, 0)),
            pl.BlockSpec((1, E), lambda t: (0, 0)),
            pl.BlockSpec((1, 1), lambda t: (0, 0)),
        ],
        out_shape=[
            jax.ShapeDtypeStruct((NT, 1, TT), jnp.int32),
            jax.ShapeDtypeStruct((NT, 1, TT), jnp.int32),
            jax.ShapeDtypeStruct((N, GW), jnp.float32),
            jax.ShapeDtypeStruct((1, E), jnp.int32),
            jax.ShapeDtypeStruct((1, 1), jnp.float32),
        ],
        scratch_shapes=[
            pltpu.VMEM((1, E), jnp.float32),
            pltpu.VMEM((1, E), jnp.float32),
            pltpu.VMEM((TT, TT), jnp.float32),
        ],
        compiler_params=pltpu.CompilerParams(
            dimension_semantics=("arbitrary",)),
    )(x_flat, Wr, br2)


@functools.lru_cache(maxsize=None)
def _dispatch_kernel():
    mesh = plsc.VectorSubcoreMesh(core_axis_name="c", subcore_axis_name="s")

    @functools.partial(
        pl.kernel,
        out_type=[
            jax.ShapeDtypeStruct((E * PAD, C), jnp.float32),
            jax.ShapeDtypeStruct((E * PAD, GW), jnp.float32),
        ],
        mesh=mesh,
        scratch_types=[
            pltpu.VMEM((CHUNK,), jnp.int32),
            pltpu.VMEM((CHUNK,), jnp.int32),
            pltpu.VMEM((CHUNK, C), jnp.float32),
            pltpu.VMEM((CHUNK, C), jnp.float32),
            pltpu.VMEM((CHUNK, GW), jnp.float32),
            pltpu.VMEM((CHUNK, GW), jnp.float32),
            pltpu.SemaphoreType.DMA,
            pltpu.SemaphoreType.DMA,
        ],
    )
    def dispatch(x_hbm, val_hbm, dst_hbm, buf_hbm, gate_hbm,
                 idx_a, idx_b, rows_a, rows_b, gv_a, gv_b, sem_a, sem_b):
        wid = lax.axis_index("s") * 2 + lax.axis_index("c")
        bufs = ((idx_a, rows_a, gv_a, sem_a), (idx_b, rows_b, gv_b, sem_b))
        n_ch = PER_W // CHUNK
        pend = [None, None]
        for cch in range(n_ch):
            idx_v, rows_v, gv_v, sem = bufs[cch % 2]
            if pend[cch % 2] is not None:
                for cp in pend[cch % 2]:
                    cp.wait()
            base = wid * PER_W + cch * CHUNK
            pltpu.sync_copy(dst_hbm.at[pl.ds(base, CHUNK)], idx_v)
            pltpu.sync_copy(x_hbm.at[pl.ds(base, CHUNK)], rows_v)
            pltpu.sync_copy(val_hbm.at[pl.ds(base, CHUNK)], gv_v)
            row_cp = pltpu.async_copy(rows_v, buf_hbm.at[idx_v], sem)
            gate_cp = pltpu.async_copy(gv_v, gate_hbm.at[idx_v], sem)
            pend[cch % 2] = (row_cp, gate_cp)
        for p in pend:
            if p is not None:
                for cp in p:
                    cp.wait()

    return dispatch


def _dispatch(x_flat, valrep, dst):
    return _dispatch_kernel()(x_flat, valrep, dst)


def _ffn_body(cnt_ref, buf_ref, w1_ref, b1_ref, w2_ref, b2_ref, gate_ref,
              out_ref, xs_bf):
    e = pl.program_id(0)
    h = pl.program_id(1)
    cnt = cnt_ref[0, e]

    @pl.when(h == 0)
    def _():
        out_ref[0, CAP:PAD, :] = jnp.zeros((PAD - CAP, C), jnp.float32)
        xs_bf[...] = buf_ref[0].astype(jnp.bfloat16)

    w1 = w1_ref[0].astype(jnp.bfloat16)
    w2 = w2_ref[0].astype(jnp.bfloat16)
    for st, sz in SUBS:
        @pl.when(st < cnt)
        def _():
            xs = xs_bf[st:st + sz, :]
            hs = jnp.dot(xs, w1, preferred_element_type=jnp.float32)
            hs = jnp.maximum(hs + b1_ref[0], 0.0).astype(jnp.bfloat16)
            contrib = jnp.dot(hs, w2, preferred_element_type=jnp.float32)

            @pl.when(h == 0)
            def _():
                out_ref[0, st:st + sz, :] = contrib

            @pl.when(h > 0)
            def _():
                g = gate_ref[0, st:st + sz, 0:1]
                out_ref[0, st:st + sz, :] = (
                    out_ref[0, st:st + sz, :] + contrib + b2_ref[0]) * g


def _ffn(cnt, buf, W1, b1, W2, b2, gate):
    return pl.pallas_call(
        _ffn_body,
        grid=(E, NH),
        in_specs=[
            pl.BlockSpec(memory_space=pltpu.SMEM),
            pl.BlockSpec((1, PAD, C), lambda e, h: (e, 0, 0)),
            pl.BlockSpec((1, C, TH), lambda e, h: (e, 0, h)),
            pl.BlockSpec((1, 1, TH), lambda e, h: (e * NH + h, 0, 0)),
            pl.BlockSpec((1, TH, C), lambda e, h: (e, h, 0)),
            pl.BlockSpec((1, 1, C), lambda e, h: (e, 0, 0)),
            pl.BlockSpec((1, PAD, GW), lambda e, h: (e, 0, 0)),
        ],
        out_specs=pl.BlockSpec((1, PAD, C), lambda e, h: (e, 0, 0)),
        out_shape=jax.ShapeDtypeStruct((E, PAD, C), jnp.float32),
        scratch_shapes=[pltpu.VMEM((PAD, C), jnp.bfloat16)],
        compiler_params=pltpu.CompilerParams(
            dimension_semantics=("arbitrary", "arbitrary"),
            vmem_limit_bytes=62 * 1024 * 1024),
    )(cnt, buf, W1, b1.reshape(E * NH, 1, TH), W2, b2.reshape(E, 1, C), gate)


@functools.lru_cache(maxsize=None)
def _gather_kernel():
    mesh = plsc.VectorSubcoreMesh(core_axis_name="c", subcore_axis_name="s")

    @functools.partial(
        pl.kernel,
        out_type=jax.ShapeDtypeStruct((N, C), jnp.float32),
        mesh=mesh,
        scratch_types=[
            pltpu.VMEM((CHUNK,), jnp.int32),
            pltpu.VMEM((CHUNK,), jnp.int32),
            pltpu.VMEM((CHUNK, C), jnp.float32),
            pltpu.VMEM((CHUNK, C), jnp.float32),
            pltpu.SemaphoreType.DMA,
            pltpu.SemaphoreType.DMA,
        ],
    )
    def gather(outbuf_hbm, gdst_hbm, raw_hbm,
               idx_a, idx_b, rows_a, rows_b, sem_a, sem_b):
        wid = lax.axis_index("s") * 2 + lax.axis_index("c")
        bufs = ((idx_a, rows_a, sem_a), (idx_b, rows_b, sem_b))
        n_ch = PER_W // CHUNK
        pend = [None, None]
        for cch in range(n_ch):
            idx_v, rows_v, sem = bufs[cch % 2]
            if pend[cch % 2] is not None:
                pbase, pcp = pend[cch % 2]
                pcp.wait()
                pltpu.sync_copy(rows_v, raw_hbm.at[pl.ds(pbase, CHUNK)])
            base = wid * PER_W + cch * CHUNK
            pltpu.sync_copy(gdst_hbm.at[pl.ds(base, CHUNK)], idx_v)
            pend[cch % 2] = (base, pltpu.async_copy(
                outbuf_hbm.at[idx_v], rows_v, sem))
        for par, (idx_v, rows_v, sem) in zip(pend, bufs):
            if par is not None:
                pbase, pcp = par
                pcp.wait()
                pltpu.sync_copy(rows_v, raw_hbm.at[pl.ds(pbase, CHUNK)])

    return gather


def _gather(outbuf, gdst):
    return _gather_kernel()(outbuf, gdst)


def kernel(x, Wr, br, W1, b1, W2, b2):
    B, T, _ = x.shape
    x_flat = x.reshape(N, C)
    dst3, gdst3, valrep, cnt, aux = _router(x_flat, Wr, br.reshape(1, E))
    dst = dst3.reshape(N)
    gdst = gdst3.reshape(N)
    buf, gate = _dispatch(x_flat, valrep, dst)
    outbuf = _ffn(cnt, buf.reshape(E, PAD, C), W1, b1, W2, b2,
                  gate.reshape(E, PAD, GW))
    out = _gather(outbuf.reshape(E * PAD, C), gdst).reshape(B, T, C)
    return out, aux[0, 0]


# speedup vs baseline: 1.0223x; 1.0223x over previous
import functools

import jax
import jax.numpy as jnp
from jax import lax
from jax.experimental import pallas as pl
from jax.experimental.pallas import tpu as pltpu
from jax.experimental.pallas import tpu_sc as plsc

C = 1024
E = 8
HID = 4096
N = 4096
CAP = 640
PAD = 768
TT = 512
NT = N // TT
TH = 2048
NH = HID // TH
SUBS = ((0, 256), (256, 256), (512, 128))

CHUNK = 64
NW = 32
PER_W = N // NW
GW = 128


def _router_body(x_ref, wr_ref, br_ref,
                 dst_ref, gdst_ref, val_ref, cnt_ref, aux_ref,
                 cnt_acc, imp_acc, tril_s):
    t = pl.program_id(0)

    @pl.when(t == 0)
    def _():
        cnt_acc[...] = jnp.zeros_like(cnt_acc)
        imp_acc[...] = jnp.zeros_like(imp_acc)
        ri = lax.broadcasted_iota(jnp.int32, (TT, TT), 0)
        ci = lax.broadcasted_iota(jnp.int32, (TT, TT), 1)
        tril_s[...] = (ri > ci).astype(jnp.float32)

    logits = jnp.dot(x_ref[...], wr_ref[...],
                     preferred_element_type=jnp.float32) + br_ref[...]
    m = jnp.max(logits, axis=1, keepdims=True)
    ex = jnp.exp(logits - m)
    probs = ex / jnp.sum(ex, axis=1, keepdims=True)
    pmax = jnp.max(probs, axis=1, keepdims=True)
    eids = lax.broadcasted_iota(jnp.int32, (TT, E), 1)
    idx = jnp.min(jnp.where(probs == pmax, eids, E), axis=1)
    mask = (eids == idx[:, None]).astype(jnp.float32)

    prev = jnp.dot(tril_s[...], mask,
                   preferred_element_type=jnp.float32) + cnt_acc[...]
    pos = jnp.sum(prev * mask, axis=1).astype(jnp.int32)

    cnt_acc[...] = cnt_acc[...] + jnp.sum(mask, axis=0, keepdims=True)
    imp_acc[...] = imp_acc[...] + jnp.sum(probs, axis=0, keepdims=True)

    kept = pos < CAP
    dst_ref[...] = (idx * PAD + jnp.where(kept, pos, CAP)).reshape(1, 1, TT)
    gdst_ref[...] = (idx * PAD + jnp.where(kept, pos, CAP + 1)).reshape(1, 1, TT)
    val_ref[...] = jnp.broadcast_to(pmax, (TT, GW))

    @pl.when(t == NT - 1)
    def _():
        cnt_k = jnp.minimum(cnt_acc[...], float(CAP))
        cnt_ref[...] = cnt_k.astype(jnp.int32)
        aux = jnp.sum((imp_acc[...] / N) * (cnt_k / N)) * E
        aux_ref[...] = aux.reshape(1, 1)


def _router(x_flat, Wr, br2):
    return pl.pallas_call(
        _router_body,
        grid=(NT,),
        in_specs=[
            pl.BlockSpec((TT, C), lambda t: (t, 0)),
            pl.BlockSpec((C, E), lambda t: (0, 0)),
            pl.BlockSpec((1, E), lambda t: (0, 0)),
        ],
        out_specs=[
            pl.BlockSpec((1, 1, TT), lambda t: (t, 0, 0)),
            pl.BlockSpec((1, 1, TT), lambda t: (t, 0, 0)),
            pl.BlockSpec((TT, GW), lambda t: (t, 0)),
            pl.BlockSpec((1, E), lambda t: (0, 0)),
            pl.BlockSpec((1, 1), lambda t: (0, 0)),
        ],
        out_shape=[
            jax.ShapeDtypeStruct((NT, 1, TT), jnp.int32),
            jax.ShapeDtypeStruct((NT, 1, TT), jnp.int32),
            jax.ShapeDtypeStruct((N, GW), jnp.float32),
            jax.ShapeDtypeStruct((1, E), jnp.int32),
            jax.ShapeDtypeStruct((1, 1), jnp.float32),
        ],
        scratch_shapes=[
            pltpu.VMEM((1, E), jnp.float32),
            pltpu.VMEM((1, E), jnp.float32),
            pltpu.VMEM((TT, TT), jnp.float32),
        ],
        compiler_params=pltpu.CompilerParams(
            dimension_semantics=("arbitrary",)),
    )(x_flat, Wr, br2)


@functools.lru_cache(maxsize=None)
def _dispatch_kernel():
    mesh = plsc.VectorSubcoreMesh(core_axis_name="c", subcore_axis_name="s")

    @functools.partial(
        pl.kernel,
        out_type=[
            jax.ShapeDtypeStruct((E * PAD, C), jnp.float32),
            jax.ShapeDtypeStruct((E * PAD, GW), jnp.float32),
        ],
        mesh=mesh,
        scratch_types=[
            pltpu.VMEM((CHUNK,), jnp.int32),
            pltpu.VMEM((CHUNK, C), jnp.float32),
            pltpu.VMEM((CHUNK, GW), jnp.float32),
            pltpu.SemaphoreType.DMA,
            pltpu.SemaphoreType.DMA,
        ],
    )
    def dispatch(x_hbm, val_hbm, dst_hbm, buf_hbm, gate_hbm,
                 idx_v, rows_v, gv_v, sem, sem2):
        wid = lax.axis_index("s") * 2 + lax.axis_index("c")
        for cch in range(PER_W // CHUNK):
            base = wid * PER_W + cch * CHUNK
            pltpu.sync_copy(dst_hbm.at[pl.ds(base, CHUNK)], idx_v)
            pltpu.sync_copy(x_hbm.at[pl.ds(base, CHUNK)], rows_v)
            pltpu.sync_copy(val_hbm.at[pl.ds(base, CHUNK)], gv_v)
            row_cp = pltpu.async_copy(rows_v, buf_hbm.at[idx_v], sem)
            gate_cp = pltpu.async_copy(gv_v, gate_hbm.at[idx_v], sem2)
            row_cp.wait()
            gate_cp.wait()

    return dispatch


def _dispatch(x_flat, valrep, dst):
    return _dispatch_kernel()(x_flat, valrep, dst)


def _ffn_body(cnt_ref, buf_ref, w1_ref, b1_ref, w2_ref, b2_ref, gate_ref,
              out_ref, xs_bf):
    e = pl.program_id(0)
    h = pl.program_id(1)
    cnt = cnt_ref[0, e]

    @pl.when(h == 0)
    def _():
        out_ref[0, CAP:PAD, :] = jnp.zeros((PAD - CAP, C), jnp.float32)
        xs_bf[...] = buf_ref[0].astype(jnp.bfloat16)

    w1 = w1_ref[0].astype(jnp.bfloat16)
    w2 = w2_ref[0].astype(jnp.bfloat16)
    for st, sz in SUBS:
        @pl.when(st < cnt)
        def _():
            xs = xs_bf[st:st + sz, :]
            hs = jnp.dot(xs, w1, preferred_element_type=jnp.float32)
            hs = jnp.maximum(hs + b1_ref[0], 0.0).astype(jnp.bfloat16)
            contrib = jnp.dot(hs, w2, preferred_element_type=jnp.float32)

            @pl.when(h == 0)
            def _():
                out_ref[0, st:st + sz, :] = contrib

            @pl.when(h > 0)
            def _():
                g = gate_ref[0, st:st + sz, 0:1]
                out_ref[0, st:st + sz, :] = (
                    out_ref[0, st:st + sz, :] + contrib + b2_ref[0]) * g


def _ffn(cnt, buf, W1, b1, W2, b2, gate):
    return pl.pallas_call(
        _ffn_body,
        grid=(E, NH),
        in_specs=[
            pl.BlockSpec(memory_space=pltpu.SMEM),
            pl.BlockSpec((1, PAD, C), lambda e, h: (e, 0, 0)),
            pl.BlockSpec((1, C, TH), lambda e, h: (e, 0, h)),
            pl.BlockSpec((1, 1, TH), lambda e, h: (e * NH + h, 0, 0)),
            pl.BlockSpec((1, TH, C), lambda e, h: (e, h, 0)),
            pl.BlockSpec((1, 1, C), lambda e, h: (e, 0, 0)),
            pl.BlockSpec((1, PAD, GW), lambda e, h: (e, 0, 0)),
        ],
        out_specs=pl.BlockSpec((1, PAD, C), lambda e, h: (e, 0, 0)),
        out_shape=jax.ShapeDtypeStruct((E, PAD, C), jnp.float32),
        scratch_shapes=[pltpu.VMEM((PAD, C), jnp.bfloat16)],
        compiler_params=pltpu.CompilerParams(
            dimension_semantics=("arbitrary", "arbitrary"),
            vmem_limit_bytes=62 * 1024 * 1024),
    )(cnt, buf, W1, b1.reshape(E * NH, 1, TH), W2, b2.reshape(E, 1, C), gate)


@functools.lru_cache(maxsize=None)
def _gather_kernel():
    mesh = plsc.VectorSubcoreMesh(core_axis_name="c", subcore_axis_name="s")

    @functools.partial(
        pl.kernel,
        out_type=jax.ShapeDtypeStruct((N, C), jnp.float32),
        mesh=mesh,
        scratch_types=[
            pltpu.VMEM((CHUNK,), jnp.int32),
            pltpu.VMEM((CHUNK, C), jnp.float32),
            pltpu.SemaphoreType.DMA,
        ],
    )
    def gather(outbuf_hbm, gdst_hbm, raw_hbm, idx_v, rows_v, sem):
        wid = lax.axis_index("s") * 2 + lax.axis_index("c")
        for cch in range(PER_W // CHUNK):
            base = wid * PER_W + cch * CHUNK
            pltpu.sync_copy(gdst_hbm.at[pl.ds(base, CHUNK)], idx_v)
            pltpu.async_copy(outbuf_hbm.at[idx_v], rows_v, sem).wait()
            pltpu.sync_copy(rows_v, raw_hbm.at[pl.ds(base, CHUNK)])

    return gather


def _gather(outbuf, gdst):
    return _gather_kernel()(outbuf, gdst)


def kernel(x, Wr, br, W1, b1, W2, b2):
    B, T, _ = x.shape
    x_flat = x.reshape(N, C)
    dst3, gdst3, valrep, cnt, aux = _router(x_flat, Wr, br.reshape(1, E))
    dst = dst3.reshape(N)
    gdst = gdst3.reshape(N)
    buf, gate = _dispatch(x_flat, valrep, dst)
    outbuf = _ffn(cnt, buf.reshape(E, PAD, C), W1, b1, W2, b2,
                  gate.reshape(E, PAD, GW))
    out = _gather(outbuf.reshape(E * PAD, C), gdst).reshape(B, T, C)
    return out, aux[0, 0]
